# baseline (speedup 1.0000x reference)
import functools

import jax
import jax.numpy as jnp
from jax import lax
from jax.experimental import pallas as pl
from jax.experimental.pallas import tpu as pltpu
from jax.experimental.pallas import tpu_sc as plsc

_B = 16384
_F = 26
_V = 1000
_D = 16
_CONT = 13
_NW = 32
_CHUNK = 128
_FIRE = 13
_GROUP = _FIRE * _CHUNK

_HID1 = 858
_HID2 = 429
_EMBW = _F * _D

_NSLAB = 1
_SLAB = _B // _NSLAB


def _make_sc_gather(lookups):
    per_w = lookups // _NW
    n_ch = per_w // _CHUNK
    n_group = per_w // _GROUP
    rows = lookups // _F
    assert n_group * _GROUP == per_w

    def body(tab_hbm, idx_hbm, out_hbm, idx_v, buf_v, gsem, ssem):
        wid = lax.axis_index("s") * 2 + lax.axis_index("c")
        base = wid * per_w
        pltpu.sync_copy(idx_hbm.at[pl.ds(wid * n_ch, n_ch)], idx_v)

        def fire(g):
            descs = []
            for j in range(_FIRE):
                descs.append(pltpu.make_async_copy(
                    tab_hbm.at[idx_v.at[g * _FIRE + j]],
                    buf_v.at[g % 2, pl.ds(j * _CHUNK, _CHUNK)],
                    gsem))
            for dsc in descs:
                dsc.start()
            return descs

        def store(g):
            dsc = pltpu.make_async_copy(
                buf_v.at[g % 2], out_hbm.at[pl.ds(base + g * _GROUP, _GROUP)],
                ssem)
            dsc.start()
            return dsc

        stores = {}
        gathers = {0: fire(0)}
        for g in range(n_group):
            if g + 1 < n_group:
                if g - 1 >= 0:
                    stores[g - 1].wait()
                gathers[g + 1] = fire(g + 1)
            for dsc in gathers[g]:
                dsc.wait()
            stores[g] = store(g)
        if n_group >= 2:
            stores[n_group - 2].wait()
        stores[n_group - 1].wait()

    return pl.kernel(
        body,
        out_type=jax.ShapeDtypeStruct((lookups, _D), jnp.float32),
        mesh=plsc.VectorSubcoreMesh(core_axis_name="c", subcore_axis_name="s",
                                    num_cores=2, num_subcores=16),
        compiler_params=pltpu.CompilerParams(
            use_tc_tiling_on_sc=False,
            disable_bounds_checks=True,
            disable_semaphore_checks=True,
        ),
        scratch_types=[
            pltpu.VMEM((n_ch, _CHUNK), jnp.int32),
            pltpu.VMEM((2, _GROUP, _D), jnp.float32),
            pltpu.SemaphoreType.DMA,
            pltpu.SemaphoreType.DMA,
        ],
    )


def _mlp_body(emb_ref, xc_ref, w1a_ref, w1b_ref, b1_ref, w2_ref, b2_ref,
              w3_ref, b3_ref, o_ref):
    bf = jnp.bfloat16
    x1 = jnp.dot(emb_ref[...], w1a_ref[...],
                 preferred_element_type=jnp.float32)
    x1 = x1 + jnp.dot(xc_ref[...].astype(bf), w1b_ref[...],
                      preferred_element_type=jnp.float32)
    h1 = jnp.maximum(x1 + b1_ref[...], 0.0).astype(bf)
    h2 = jnp.maximum(
        jnp.dot(h1, w2_ref[...], preferred_element_type=jnp.float32)
        + b2_ref[...], 0.0).astype(bf)
    o_ref[...] = (jnp.dot(h2, w3_ref[...], preferred_element_type=jnp.float32)
                  + b3_ref[...])


def _mlp(emb, xc_p, w1a, w1b, b1r, w2, b2r, w3, b3r, bm=1024):
    rows = emb.shape[0]
    grid = (rows // bm,)
    full = lambda shape: pl.BlockSpec(shape, lambda i: (0, 0))
    return pl.pallas_call(
        _mlp_body,
        grid=grid,
        in_specs=[
            pl.BlockSpec((bm, _EMBW), lambda i: (i, 0)),
            pl.BlockSpec((bm, _CONT), lambda i: (i, 0)),
            full((_EMBW, _HID1)),
            full((_CONT, _HID1)),
            full((1, _HID1)),
            full((_HID1, _HID2)),
            full((1, _HID2)),
            full((_HID2, 1)),
            full((1, 1)),
        ],
        out_specs=pl.BlockSpec((bm, 1), lambda i: (i, 0)),
        out_shape=jax.ShapeDtypeStruct((rows, 1), jnp.float32),
        compiler_params=pltpu.CompilerParams(
            dimension_semantics=("arbitrary",)),
    )(emb, xc_p, w1a, w1b, b1r, w2, b2r, w3, b3r)


def kernel(x_cat, x_cont, tables, W1, b1, W2, b2, W3, b3):
    tab_flat = tables.reshape(_F * _V, _D)
    flat_idx = (x_cat.astype(jnp.int32)
                + (jnp.arange(_F, dtype=jnp.int32) * _V)[None, :])
    idx2d = flat_idx.reshape((_B * _F) // _CHUNK, _CHUNK)

    bf = jnp.bfloat16
    w1a = W1[:_EMBW].astype(bf)
    w1b = W1[_EMBW:].astype(bf)
    w2 = W2.astype(bf)
    w3 = W3.astype(bf)
    b1r = b1.reshape(1, _HID1)
    b2r = b2.reshape(1, _HID2)
    b3r = b3.reshape(1, 1)

    gather = _make_sc_gather(_SLAB * _F)
    outs = []
    for s in range(_NSLAB):
        nch = (_SLAB * _F) // _CHUNK
        emb = gather(tab_flat, idx2d[s * nch:(s + 1) * nch]).astype(
            bf).reshape(_SLAB, _EMBW)
        outs.append(_mlp(emb, x_cont[s * _SLAB:(s + 1) * _SLAB],
                         w1a, w1b, b1r, w2, b2r, w3, b3r))
    return outs[0] if _NSLAB == 1 else jnp.concatenate(outs, axis=0)


# speedup vs baseline: 2.0531x; 2.0531x over previous
import functools

import jax
import jax.numpy as jnp
from jax import lax
from jax.experimental import pallas as pl
from jax.experimental.pallas import tpu as pltpu
from jax.experimental.pallas import tpu_sc as plsc

_B = 16384
_F = 26
_V = 1000
_D = 16
_CONT = 13
_NW = 32
_CHUNK = 128
_FIRE = 13
_GROUP = _FIRE * _CHUNK

_HID1 = 858
_HID2 = 429
_EMBW = _F * _D

_NSLAB = 2
_SLAB = _B // _NSLAB


def _make_sc_gather(lookups):
    per_w = lookups // _NW
    n_ch = per_w // _CHUNK
    n_group = per_w // _GROUP
    rows = lookups // _F
    assert n_group * _GROUP == per_w

    def body(tab_hbm, idx_hbm, out_hbm, idx_v, buf_v, gsem, ssem):
        wid = lax.axis_index("s") * 2 + lax.axis_index("c")
        base = wid * per_w
        pltpu.sync_copy(idx_hbm.at[pl.ds(wid * n_ch, n_ch)], idx_v)

        def fire(g):
            descs = []
            for j in range(_FIRE):
                descs.append(pltpu.make_async_copy(
                    tab_hbm.at[idx_v.at[g * _FIRE + j]],
                    buf_v.at[g % 2, pl.ds(j * _CHUNK, _CHUNK)],
                    gsem))
            for dsc in descs:
                dsc.start()
            return descs

        def store(g):
            dsc = pltpu.make_async_copy(
                buf_v.at[g % 2], out_hbm.at[pl.ds(base + g * _GROUP, _GROUP)],
                ssem)
            dsc.start()
            return dsc

        stores = {}
        gathers = {0: fire(0)}
        for g in range(n_group):
            if g + 1 < n_group:
                if g - 1 >= 0:
                    stores[g - 1].wait()
                gathers[g + 1] = fire(g + 1)
            for dsc in gathers[g]:
                dsc.wait()
            stores[g] = store(g)
        if n_group >= 2:
            stores[n_group - 2].wait()
        stores[n_group - 1].wait()

    return pl.kernel(
        body,
        out_type=jax.ShapeDtypeStruct((lookups, _D), jnp.float32),
        mesh=plsc.VectorSubcoreMesh(core_axis_name="c", subcore_axis_name="s",
                                    num_cores=2, num_subcores=16),
        compiler_params=pltpu.CompilerParams(
            use_tc_tiling_on_sc=False,
            disable_bounds_checks=True,
            disable_semaphore_checks=True,
        ),
        scratch_types=[
            pltpu.VMEM((n_ch, _CHUNK), jnp.int32),
            pltpu.VMEM((2, _GROUP, _D), jnp.float32),
            pltpu.SemaphoreType.DMA,
            pltpu.SemaphoreType.DMA,
        ],
    )


def _mlp_body(emb_ref, xc_ref, w1a_ref, w1b_ref, b1_ref, w2_ref, b2_ref,
              w3_ref, b3_ref, o_ref):
    bf = jnp.bfloat16
    x1 = jnp.dot(emb_ref[...].astype(bf), w1a_ref[...],
                 preferred_element_type=jnp.float32)
    x1 = x1 + jnp.dot(xc_ref[...].astype(bf), w1b_ref[...],
                      preferred_element_type=jnp.float32)
    h1 = jnp.maximum(x1 + b1_ref[...], 0.0).astype(bf)
    h2 = jnp.maximum(
        jnp.dot(h1, w2_ref[...], preferred_element_type=jnp.float32)
        + b2_ref[...], 0.0).astype(bf)
    o_ref[...] = (jnp.dot(h2, w3_ref[...], preferred_element_type=jnp.float32)
                  + b3_ref[...])


def _mlp(emb, xc_p, w1a, w1b, b1r, w2, b2r, w3, b3r, bm=1024):
    rows = emb.shape[0]
    grid = (rows // bm,)
    full = lambda shape: pl.BlockSpec(shape, lambda i: (0, 0))
    return pl.pallas_call(
        _mlp_body,
        grid=grid,
        in_specs=[
            pl.BlockSpec((bm, _EMBW), lambda i: (i, 0)),
            pl.BlockSpec((bm, _CONT), lambda i: (i, 0)),
            full((_EMBW, _HID1)),
            full((_CONT, _HID1)),
            full((1, _HID1)),
            full((_HID1, _HID2)),
            full((1, _HID2)),
            full((_HID2, 1)),
            full((1, 1)),
        ],
        out_specs=pl.BlockSpec((bm, 1), lambda i: (i, 0)),
        out_shape=jax.ShapeDtypeStruct((rows, 1), jnp.float32),
        compiler_params=pltpu.CompilerParams(
            dimension_semantics=("arbitrary",)),
    )(emb, xc_p, w1a, w1b, b1r, w2, b2r, w3, b3r)


def kernel(x_cat, x_cont, tables, W1, b1, W2, b2, W3, b3):
    tab_flat = tables.reshape(_F * _V, _D)
    flat_idx = (x_cat.astype(jnp.int32)
                + (jnp.arange(_F, dtype=jnp.int32) * _V)[None, :])
    idx2d = flat_idx.reshape((_B * _F) // _CHUNK, _CHUNK)

    bf = jnp.bfloat16
    w1a = W1[:_EMBW].astype(bf)
    w1b = W1[_EMBW:].astype(bf)
    w2 = W2.astype(bf)
    w3 = W3.astype(bf)
    b1r = b1.reshape(1, _HID1)
    b2r = b2.reshape(1, _HID2)
    b3r = b3.reshape(1, 1)

    gather = _make_sc_gather(_SLAB * _F)
    outs = []
    for s in range(_NSLAB):
        nch = (_SLAB * _F) // _CHUNK
        emb = gather(tab_flat, idx2d[s * nch:(s + 1) * nch]).reshape(
            _SLAB, _EMBW)
        outs.append(_mlp(emb, x_cont[s * _SLAB:(s + 1) * _SLAB],
                         w1a, w1b, b1r, w2, b2r, w3, b3r))
    return outs[0] if _NSLAB == 1 else jnp.concatenate(outs, axis=0)


# speedup vs baseline: 2.1091x; 1.0273x over previous
import functools

import jax
import jax.numpy as jnp
from jax import lax
from jax.experimental import pallas as pl
from jax.experimental.pallas import tpu as pltpu
from jax.experimental.pallas import tpu_sc as plsc

_B = 16384
_F = 26
_V = 1000
_D = 16
_CONT = 13
_NW = 32
_CHUNK = 128
_FIRE = 13
_GROUP = _FIRE * _CHUNK

_HID1 = 858
_HID2 = 429
_EMBW = _F * _D

_NSLAB = 1
_SLAB = _B // _NSLAB


def _make_sc_gather(lookups):
    per_w = lookups // _NW
    n_ch = per_w // _CHUNK
    n_group = per_w // _GROUP
    rows = lookups // _F
    assert n_group * _GROUP == per_w

    def body(tab_hbm, idx_hbm, out_hbm, idx_v, buf_v, gsem, ssem):
        wid = lax.axis_index("s") * 2 + lax.axis_index("c")
        base = wid * per_w
        pltpu.sync_copy(idx_hbm.at[pl.ds(wid * n_ch, n_ch)], idx_v)

        def fire(g):
            descs = []
            for j in range(_FIRE):
                descs.append(pltpu.make_async_copy(
                    tab_hbm.at[idx_v.at[g * _FIRE + j]],
                    buf_v.at[g % 2, pl.ds(j * _CHUNK, _CHUNK)],
                    gsem))
            for dsc in descs:
                dsc.start()
            return descs

        def store(g):
            dsc = pltpu.make_async_copy(
                buf_v.at[g % 2], out_hbm.at[pl.ds(base + g * _GROUP, _GROUP)],
                ssem)
            dsc.start()
            return dsc

        stores = {}
        gathers = {0: fire(0)}
        for g in range(n_group):
            if g + 1 < n_group:
                if g - 1 >= 0:
                    stores[g - 1].wait()
                gathers[g + 1] = fire(g + 1)
            for dsc in gathers[g]:
                dsc.wait()
            stores[g] = store(g)
        if n_group >= 2:
            stores[n_group - 2].wait()
        stores[n_group - 1].wait()

    return pl.kernel(
        body,
        out_type=jax.ShapeDtypeStruct((lookups, _D), jnp.float32),
        mesh=plsc.VectorSubcoreMesh(core_axis_name="c", subcore_axis_name="s",
                                    num_cores=2, num_subcores=16),
        compiler_params=pltpu.CompilerParams(
            use_tc_tiling_on_sc=False,
            disable_bounds_checks=True,
            disable_semaphore_checks=True,
        ),
        scratch_types=[
            pltpu.VMEM((n_ch, _CHUNK), jnp.int32),
            pltpu.VMEM((2, _GROUP, _D), jnp.float32),
            pltpu.SemaphoreType.DMA,
            pltpu.SemaphoreType.DMA,
        ],
    )


def _mlp_body(emb_ref, xc_ref, w1a_ref, w1b_ref, b1_ref, w2_ref, b2_ref,
              w3_ref, b3_ref, o_ref):
    bf = jnp.bfloat16
    x1 = jnp.dot(emb_ref[...].astype(bf), w1a_ref[...],
                 preferred_element_type=jnp.float32)
    x1 = x1 + jnp.dot(xc_ref[...].astype(bf), w1b_ref[...],
                      preferred_element_type=jnp.float32)
    h1 = jnp.maximum(x1 + b1_ref[...], 0.0).astype(bf)
    h2 = jnp.maximum(
        jnp.dot(h1, w2_ref[...], preferred_element_type=jnp.float32)
        + b2_ref[...], 0.0).astype(bf)
    o_ref[...] = (jnp.dot(h2, w3_ref[...], preferred_element_type=jnp.float32)
                  + b3_ref[...])


def _mlp(emb, xc_p, w1a, w1b, b1r, w2, b2r, w3, b3r, bm=2048):
    rows = emb.shape[0]
    grid = (rows // bm,)
    full = lambda shape: pl.BlockSpec(shape, lambda i: (0, 0))
    return pl.pallas_call(
        _mlp_body,
        grid=grid,
        in_specs=[
            pl.BlockSpec((bm, _EMBW), lambda i: (i, 0)),
            pl.BlockSpec((bm, _CONT), lambda i: (i, 0)),
            full((_EMBW, _HID1)),
            full((_CONT, _HID1)),
            full((1, _HID1)),
            full((_HID1, _HID2)),
            full((1, _HID2)),
            full((_HID2, 1)),
            full((1, 1)),
        ],
        out_specs=pl.BlockSpec((bm, 1), lambda i: (i, 0)),
        out_shape=jax.ShapeDtypeStruct((rows, 1), jnp.float32),
        compiler_params=pltpu.CompilerParams(
            dimension_semantics=("arbitrary",)),
    )(emb, xc_p, w1a, w1b, b1r, w2, b2r, w3, b3r)


def kernel(x_cat, x_cont, tables, W1, b1, W2, b2, W3, b3):
    tab_flat = tables.reshape(_F * _V, _D)
    flat_idx = (x_cat.astype(jnp.int32)
                + (jnp.arange(_F, dtype=jnp.int32) * _V)[None, :])
    idx2d = flat_idx.reshape((_B * _F) // _CHUNK, _CHUNK)

    bf = jnp.bfloat16
    w1a = W1[:_EMBW].astype(bf)
    w1b = W1[_EMBW:].astype(bf)
    w2 = W2.astype(bf)
    w3 = W3.astype(bf)
    b1r = b1.reshape(1, _HID1)
    b2r = b2.reshape(1, _HID2)
    b3r = b3.reshape(1, 1)

    gather = _make_sc_gather(_SLAB * _F)
    outs = []
    for s in range(_NSLAB):
        nch = (_SLAB * _F) // _CHUNK
        emb = gather(tab_flat, idx2d[s * nch:(s + 1) * nch]).reshape(
            _SLAB, _EMBW)
        outs.append(_mlp(emb, x_cont[s * _SLAB:(s + 1) * _SLAB],
                         w1a, w1b, b1r, w2, b2r, w3, b3r))
    return outs[0] if _NSLAB == 1 else jnp.concatenate(outs, axis=0)
